# baseline (speedup 1.0000x reference)
import dataclasses
import functools

import jax
import jax.numpy as jnp
from jax import lax
from jax.experimental import pallas as pl
from jax.experimental.pallas import tpu as pltpu
from jax.experimental.pallas import tpu_sc as plsc

_EMBED = 384
_NQ = 32
_K = 8
_CORPUS = 100000
_BLK = 4096
_PAD = 102400
_NB = _PAD // _BLK
_CH = 12800
_NCH = _PAD // _CH
_GRP = 8
_NGRP = _CH // (16 * _GRP)


def _score_kernel(q_ref, k_ref, s_ref):
    i = pl.program_id(0)
    kb = k_ref[...]
    norm = jnp.sqrt(jnp.sum(kb * kb, axis=1, keepdims=True))
    kn = kb * (1.0 / jnp.maximum(norm, 1e-12))
    s = jax.lax.dot_general(
        q_ref[...], kn.astype(jnp.bfloat16),
        (((1,), (1,)), ((), ())),
        preferred_element_type=jnp.float32,
    )

    @pl.when(i < _NB - 1)
    def _store():
        s_ref[...] = s

    @pl.when(i == _NB - 1)
    def _store_masked():
        lane = jax.lax.broadcasted_iota(jnp.int32, (_NQ, _BLK), 1)
        s_ref[...] = jnp.where(lane + i * _BLK < _CORPUS, s, -jnp.inf)


_vsc_mesh = plsc.VectorSubcoreMesh(core_axis_name="c", subcore_axis_name="s")

_sc_params = pltpu.CompilerParams()
if "needs_layout_passes" in pltpu.CompilerParams.__dataclass_fields__:
    _sc_params = dataclasses.replace(_sc_params, needs_layout_passes=False)


@functools.partial(
    pl.kernel,
    compiler_params=_sc_params,
    out_type=[
        jax.ShapeDtypeStruct((_NQ, 16), jnp.float32),
        jax.ShapeDtypeStruct((_NQ, 16), jnp.int32),
    ],
    mesh=_vsc_mesh,
    scratch_types=[
        pltpu.VMEM((_CH,), jnp.float32),
        pltpu.VMEM((_CH,), jnp.float32),
        pltpu.VMEM((16,), jnp.float32),
        pltpu.VMEM((16,), jnp.int32),
        pltpu.SemaphoreType.DMA,
        pltpu.SemaphoreType.DMA,
    ],
)
def _sc_topk(s_hbm, outs_hbm, outi_hbm, buf0, buf1, ovs, ovi, sem0, sem1):
    row = lax.axis_index("s") * 2 + lax.axis_index("c")
    bufs = (buf0, buf1)
    sems = (sem0, sem1)
    lane16 = lax.iota(jnp.int32, 16)
    neg_inf = jnp.float32(-jnp.inf)

    def copy(c, b):
        return pltpu.make_async_copy(
            s_hbm.at[row, pl.ds(c * _CH, _CH)], bufs[b], sems[b])

    copy(0, 0).start()
    copy(1, 1).start()

    def merge(carry_bs_bi, v, gbase):
        bs, bi = carry_bs_bi
        idx = jnp.full((16,), gbase, jnp.int32) + lane16
        sv, si = plsc.sort_key_val(v, idx, descending=True)
        rs = lax.rev(sv, (0,))
        ri = lax.rev(si, (0,))
        take = bs >= rs
        ck = jnp.where(take, bs, rs)
        ci = jnp.where(take, bi, ri)
        return plsc.sort_key_val(ck, ci, descending=True)

    def make_group_body(b, base0):
        def group_body(g, carry):
            best_s, best_i, th = carry
            base = g * (16 * _GRP)
            v = [bufs[b][pl.ds(base + 16 * j, 16)] for j in range(_GRP)]
            m01 = jnp.maximum(v[0], v[1])
            m23 = jnp.maximum(v[2], v[3])
            m45 = jnp.maximum(v[4], v[5])
            m67 = jnp.maximum(v[6], v[7])
            m = jnp.maximum(jnp.maximum(m01, m23), jnp.maximum(m45, m67))
            gm = jnp.max(m)

            def slow():
                bsbi = (best_s, best_i)
                for j in range(_GRP):
                    bsbi = merge(bsbi, v[j], base0 + base + 16 * j)
                nbs, nbi = bsbi
                nth = jnp.max(jnp.where(lane16 == _K - 1, nbs, neg_inf))
                return nbs, nbi, nth

            return lax.cond(gm > th, slow, lambda: carry)

        return group_body

    carry = (
        jnp.full((16,), -jnp.inf, jnp.float32),
        jnp.zeros((16,), jnp.int32),
        neg_inf,
    )
    for c in range(_NCH):
        b = c % 2
        copy(c, b).wait()
        carry = lax.fori_loop(0, _NGRP, make_group_body(b, c * _CH), carry)
        if c + 2 < _NCH:
            copy(c + 2, b).start()

    best_s, best_i, _ = carry
    ovs[...] = best_s
    ovi[...] = best_i
    pltpu.sync_copy(ovs, outs_hbm.at[row])
    pltpu.sync_copy(ovi, outi_hbm.at[row])


@jax.jit
def kernel(queries, keys):
    qn = queries / jnp.clip(
        jnp.linalg.norm(queries, axis=1, keepdims=True), 1e-12, None
    )
    scores = pl.pallas_call(
        _score_kernel,
        grid=(_NB,),
        in_specs=[
            pl.BlockSpec((_NQ, _EMBED), lambda i: (0, 0)),
            pl.BlockSpec((_BLK, _EMBED), lambda i: (i, 0)),
        ],
        out_specs=pl.BlockSpec((_NQ, _BLK), lambda i: (0, i)),
        out_shape=jax.ShapeDtypeStruct((_NQ, _PAD), jnp.float32),
    )(qn.astype(jnp.bfloat16), keys)
    outs16, outi16 = _sc_topk(scores)
    return outs16[:, :_K], outi16[:, :_K]
